# baseline (speedup 1.0000x reference)
import functools

import jax
import jax.numpy as jnp
from jax import lax
from jax.experimental import pallas as pl
from jax.experimental.pallas import tpu as pltpu
from jax.experimental.pallas import tpu_sc as plsc

B, L, D = 4, 8192, 768
NC, NS, LANES = 2, 16, 16
NW = NC * NS
ROWS_W = L // NW
T = 8
STEPS = ROWS_W // T
NB = 3


def _sc_body(x_hbm, t_hbm, o_hbm, t0, t1, t2, x0, x1, x2,
             s_t0, s_t1, s_t2, s_xi0, s_xi1, s_xi2, s_xo0, s_xo1, s_xo2):
    t_bufs, x_bufs = (t0, t1, t2), (x0, x1, x2)
    s_t, s_xi, s_xo = (s_t0, s_t1, s_t2), (s_xi0, s_xi1, s_xi2), (s_xo0,
                                                                  s_xo1,
                                                                  s_xo2)

    wid = lax.axis_index("s") * NC + lax.axis_index("c")
    row_at = lambda s: wid * ROWS_W + s * T

    def add_quad(xq, t_v):
        @plsc.parallel_loop(0, T)
        def _(r):
            @plsc.parallel_loop(0, D, step=LANES, unroll=4)
            def _(c):
                tv = t_v[r, pl.ds(c, LANES)]
                for bb in range(B):
                    plsc.addupdate(xq.at[bb, r, pl.ds(c, LANES)], tv)

    def start_in(slot, s):
        tin[slot] = pltpu.async_copy(
            t_hbm.at[pl.ds(row_at(s), T)], t_bufs[slot], s_t[slot])
        xin[slot] = [
            pltpu.async_copy(x_hbm.at[bb, pl.ds(row_at(s), T)],
                             x_bufs[slot].at[bb], s_xi[slot])
            for bb in range(B)
        ]

    def start_out(slot, s):
        xout[slot] = [
            pltpu.async_copy(x_bufs[slot].at[bb],
                             o_hbm.at[bb, pl.ds(row_at(s), T)], s_xo[slot])
            for bb in range(B)
        ]

    tin = [None] * NB
    xin = [None] * NB
    xout = [None] * NB
    for p in range(NB - 1):
        start_in(p, p)

    for s in range(STEPS):
        cur = s % NB
        if s + NB - 1 < STEPS:
            nxt = (s + NB - 1) % NB
            if xout[nxt] is not None:
                for d in xout[nxt]:
                    d.wait()
            start_in(nxt, s + NB - 1)
        tin[cur].wait()
        for d in xin[cur]:
            d.wait()
        add_quad(x_bufs[cur], t_bufs[cur])
        start_out(cur, s)

    for k in range(min(NB, STEPS)):
        for d in xout[(STEPS - 1 - k) % NB]:
            d.wait()


@functools.partial(
    pl.kernel,
    out_type=jax.ShapeDtypeStruct((B, L, D), jnp.float32),
    mesh=plsc.VectorSubcoreMesh(core_axis_name="c", subcore_axis_name="s"),
    scratch_types=[
        pltpu.VMEM((T, D), jnp.float32),
        pltpu.VMEM((T, D), jnp.float32),
        pltpu.VMEM((T, D), jnp.float32),
        pltpu.VMEM((B, T, D), jnp.float32),
        pltpu.VMEM((B, T, D), jnp.float32),
        pltpu.VMEM((B, T, D), jnp.float32),
        pltpu.SemaphoreType.DMA,
        pltpu.SemaphoreType.DMA,
        pltpu.SemaphoreType.DMA,
        pltpu.SemaphoreType.DMA,
        pltpu.SemaphoreType.DMA,
        pltpu.SemaphoreType.DMA,
        pltpu.SemaphoreType.DMA,
        pltpu.SemaphoreType.DMA,
        pltpu.SemaphoreType.DMA,
    ],
)
def _sc_add(*refs):
    _sc_body(*refs)


def kernel(x, table):
    return _sc_add(x, table)
